# baseline (speedup 1.0000x reference)
import functools

import jax
import jax.numpy as jnp
from jax import lax
from jax.experimental import pallas as pl
from jax.experimental.pallas import tpu as pltpu
from jax.experimental.pallas import tpu_sc as plsc

_DIM = 768
_NUM_OUT = 64
_BM = 4096

_NEG_INF = float("-inf")

_NC = 2
_NS = 16
_L = 16


def _matmul_block(x_ref, w_ref, b_ref, logits_ref):
    logits_ref[...] = (
        jax.lax.dot_general(
            w_ref[...], x_ref[...], (((1,), (1,)), ((), ())),
            preferred_element_type=jnp.float32,
        )
        + b_ref[...]
    )


def _tc_logits(input, W, b2d):
    n_tok = input.shape[0]
    return pl.pallas_call(
        _matmul_block,
        grid=(n_tok // _BM,),
        in_specs=[
            pl.BlockSpec((_BM, _DIM), lambda i: (i, 0)),
            pl.BlockSpec((_NUM_OUT, _DIM), lambda i: (0, 0)),
            pl.BlockSpec((_NUM_OUT, 1), lambda i: (0, 0)),
        ],
        out_specs=pl.BlockSpec((_NUM_OUT, _BM), lambda i: (0, i)),
        out_shape=jax.ShapeDtypeStruct((_NUM_OUT, n_tok), jnp.float32),
        compiler_params=pltpu.CompilerParams(
            dimension_semantics=("parallel",),
        ),
    )(input, W, b2d)


def _sc_top2(logits_t):
    n_tok = logits_t.shape[1]
    chunk = n_tok // (_NC * _NS)
    mesh = plsc.VectorSubcoreMesh(core_axis_name="c", subcore_axis_name="s")

    @functools.partial(
        pl.kernel,
        mesh=mesh,
        out_type=[
            jax.ShapeDtypeStruct((2, n_tok), jnp.float32),
            jax.ShapeDtypeStruct((2, n_tok), jnp.int32),
        ],
        scratch_types=[
            pltpu.VMEM((_NUM_OUT, chunk), jnp.float32),
            pltpu.VMEM((2, chunk), jnp.float32),
            pltpu.VMEM((2, chunk), jnp.int32),
        ],
    )
    def sc_kernel(logits_hbm, probs_hbm, idx_hbm, buf, pbuf, ibuf):
        wid = lax.axis_index("s") * _NC + lax.axis_index("c")
        base = wid * chunk
        pltpu.sync_copy(logits_hbm.at[:, pl.ds(base, chunk)], buf)

        def tloop(ti, carry):
            t = ti * _L
            neg = jnp.full((_L,), _NEG_INF, jnp.float32)
            zero = jnp.zeros((_L,), jnp.int32)

            def eloop(e, ecarry):
                m1, i1, m2, i2 = ecarry
                v = buf[e, pl.ds(t, _L)]
                ev = jnp.full((_L,), e, jnp.int32)
                gt1 = v > m1
                gt2 = v > m2
                m2n = jnp.where(gt1, m1, jnp.where(gt2, v, m2))
                i2n = jnp.where(gt1, i1, jnp.where(gt2, ev, i2))
                m1n = jnp.where(gt1, v, m1)
                i1n = jnp.where(gt1, ev, i1)
                return m1n, i1n, m2n, i2n

            m1, i1, m2, i2 = lax.fori_loop(
                0, _NUM_OUT, eloop, (neg, zero, neg, zero)
            )
            u = jnp.exp(m2 - m1)
            denom = 1.0 + u
            pbuf[0, pl.ds(t, _L)] = 1.0 / denom
            pbuf[1, pl.ds(t, _L)] = u / denom
            ibuf[0, pl.ds(t, _L)] = i1
            ibuf[1, pl.ds(t, _L)] = i2
            return carry

        lax.fori_loop(0, chunk // _L, tloop, 0)
        pltpu.sync_copy(pbuf, probs_hbm.at[:, pl.ds(base, chunk)])
        pltpu.sync_copy(ibuf, idx_hbm.at[:, pl.ds(base, chunk)])

    return sc_kernel(logits_t)


def kernel(input, W, b):
    b2d = b.reshape(_NUM_OUT, 1)
    logits_t = _tc_logits(input, W, b2d)
    probs_t, idx_t = _sc_top2(logits_t)
    return probs_t.T, idx_t.T


# speedup vs baseline: 1.1279x; 1.1279x over previous
import functools

import jax
import jax.numpy as jnp
from jax import lax
from jax.experimental import pallas as pl
from jax.experimental.pallas import tpu as pltpu
from jax.experimental.pallas import tpu_sc as plsc

_DIM = 768
_NUM_OUT = 64
_BM = 4096

_NEG_INF = float("-inf")

_NC = 2
_NS = 16
_L = 16


def _matmul_block(x_ref, w_ref, b_ref, logits_ref):
    logits_ref[...] = (
        jax.lax.dot_general(
            w_ref[...], x_ref[...], (((1,), (1,)), ((), ())),
            preferred_element_type=jnp.float32,
        )
        + b_ref[...]
    )


def _tc_logits(input, W, b2d):
    n_tok = input.shape[0]
    return pl.pallas_call(
        _matmul_block,
        grid=(n_tok // _BM,),
        in_specs=[
            pl.BlockSpec((_BM, _DIM), lambda i: (i, 0)),
            pl.BlockSpec((_NUM_OUT, _DIM), lambda i: (0, 0)),
            pl.BlockSpec((_NUM_OUT, 1), lambda i: (0, 0)),
        ],
        out_specs=pl.BlockSpec((_NUM_OUT, _BM), lambda i: (0, i)),
        out_shape=jax.ShapeDtypeStruct((_NUM_OUT, n_tok), jnp.float32),
        compiler_params=pltpu.CompilerParams(
            dimension_semantics=("parallel",),
        ),
    )(input, W, b2d)


def _sc_top2(logits_t):
    n_tok = logits_t.shape[1]
    chunk = n_tok // (_NC * _NS)
    mesh = plsc.VectorSubcoreMesh(core_axis_name="c", subcore_axis_name="s")

    @functools.partial(
        pl.kernel,
        mesh=mesh,
        out_type=[
            jax.ShapeDtypeStruct((2, n_tok), jnp.float32),
            jax.ShapeDtypeStruct((2, n_tok), jnp.int32),
        ],
        scratch_types=[
            pltpu.VMEM((_NUM_OUT, chunk), jnp.float32),
            pltpu.VMEM((2, chunk), jnp.float32),
            pltpu.VMEM((2, chunk), jnp.int32),
        ],
    )
    def sc_kernel(logits_hbm, probs_hbm, idx_hbm, buf, pbuf, ibuf):
        wid = lax.axis_index("s") * _NC + lax.axis_index("c")
        base = wid * chunk
        pltpu.sync_copy(logits_hbm.at[:, pl.ds(base, chunk)], buf)

        def tloop(ti, carry):
            t = ti * _L
            neg = jnp.full((_L,), _NEG_INF, jnp.float32)
            zero = jnp.zeros((_L,), jnp.int32)

            m1, i1, m2, i2 = neg, zero, neg, zero
            for e in range(_NUM_OUT):
                v = buf[e, pl.ds(t, _L)]
                ev = jnp.full((_L,), e, jnp.int32)
                gt1 = v > m1
                gt2 = v > m2
                m2, i2 = (
                    jnp.where(gt1, m1, jnp.where(gt2, v, m2)),
                    jnp.where(gt1, i1, jnp.where(gt2, ev, i2)),
                )
                m1 = jnp.where(gt1, v, m1)
                i1 = jnp.where(gt1, ev, i1)
            u = jnp.exp(m2 - m1)
            denom = 1.0 + u
            pbuf[0, pl.ds(t, _L)] = 1.0 / denom
            pbuf[1, pl.ds(t, _L)] = u / denom
            ibuf[0, pl.ds(t, _L)] = i1
            ibuf[1, pl.ds(t, _L)] = i2
            return carry

        lax.fori_loop(0, chunk // _L, tloop, 0)
        pltpu.sync_copy(pbuf, probs_hbm.at[:, pl.ds(base, chunk)])
        pltpu.sync_copy(ibuf, idx_hbm.at[:, pl.ds(base, chunk)])

    return sc_kernel(logits_t)


def kernel(input, W, b):
    b2d = b.reshape(_NUM_OUT, 1)
    logits_t = _tc_logits(input, W, b2d)
    probs_t, idx_t = _sc_top2(logits_t)
    return probs_t.T, idx_t.T


# speedup vs baseline: 1.9160x; 1.6987x over previous
import jax
import jax.numpy as jnp
from jax.experimental import pallas as pl
from jax.experimental.pallas import tpu as pltpu

_DIM = 768
_NUM_OUT = 64
_BM = 4096

_NEG_INF = float("-inf")


def _router_block(x_ref, w_ref, b_ref, out_ref):
    x = x_ref[...]
    w = w_ref[...]
    logits = jax.lax.dot_general(
        w, x, (((1,), (1,)), ((), ())), preferred_element_type=jnp.float32
    )
    logits = logits + b_ref[...]

    iota = jax.lax.broadcasted_iota(jnp.int32, logits.shape, 0).astype(jnp.float32)
    big = float(_NUM_OUT)

    v1 = jnp.max(logits, axis=0, keepdims=True)
    i1f = jnp.min(jnp.where(logits == v1, iota, big), axis=0, keepdims=True)
    masked = jnp.where(iota == i1f, _NEG_INF, logits)
    v2 = jnp.max(masked, axis=0, keepdims=True)
    i2f = jnp.min(jnp.where(masked == v2, iota, big), axis=0, keepdims=True)

    t = jnp.exp(v2 - v1)
    denom = 1.0 + t
    probs = jnp.concatenate([1.0 / denom, t / denom], axis=0)
    idx = jnp.concatenate([i1f.astype(jnp.int32), i2f.astype(jnp.int32)], axis=0)
    out_ref[...] = jnp.concatenate(
        [jax.lax.bitcast_convert_type(probs, jnp.int32), idx], axis=0
    )


def kernel(input, W, b):
    n_tok = input.shape[0]
    grid = (n_tok // _BM,)
    b2d = b.reshape(_NUM_OUT, 1)
    packed = pl.pallas_call(
        _router_block,
        grid=grid,
        in_specs=[
            pl.BlockSpec((_BM, _DIM), lambda i: (i, 0)),
            pl.BlockSpec((_NUM_OUT, _DIM), lambda i: (0, 0)),
            pl.BlockSpec((_NUM_OUT, 1), lambda i: (0, 0)),
        ],
        out_specs=pl.BlockSpec((4, _BM), lambda i: (0, i)),
        out_shape=jax.ShapeDtypeStruct((4, n_tok), jnp.int32),
        compiler_params=pltpu.CompilerParams(
            dimension_semantics=("parallel",),
        ),
    )(input, W, b2d)
    packed_t = packed.T
    probs = jax.lax.bitcast_convert_type(packed_t[:, :2], jnp.float32)
    return probs, packed_t[:, 2:]


# speedup vs baseline: 2.1143x; 1.1035x over previous
import jax
import jax.numpy as jnp
from jax.experimental import pallas as pl
from jax.experimental.pallas import tpu as pltpu

_DIM = 768
_NUM_OUT = 64
_BM = 4096

_NEG_INF = float("-inf")


def _router_block(x_ref, w_ref, b_ref, probs_ref, idx_ref):
    x = x_ref[...]
    w = w_ref[...]
    logits = jax.lax.dot_general(
        w, x, (((1,), (1,)), ((), ())), preferred_element_type=jnp.float32
    )
    logits = logits + b_ref[...]

    iota = jax.lax.broadcasted_iota(jnp.int32, logits.shape, 0).astype(jnp.float32)
    big = float(_NUM_OUT)

    v1 = jnp.max(logits, axis=0, keepdims=True)
    i1f = jnp.min(jnp.where(logits == v1, iota, big), axis=0, keepdims=True)
    masked = jnp.where(iota == i1f, _NEG_INF, logits)
    v2 = jnp.max(masked, axis=0, keepdims=True)
    i2f = jnp.min(jnp.where(masked == v2, iota, big), axis=0, keepdims=True)

    t = jnp.exp(v2 - v1)
    denom = 1.0 + t
    probs_ref[...] = jnp.concatenate([1.0 / denom, t / denom], axis=0)
    idx_ref[...] = jnp.concatenate(
        [i1f.astype(jnp.int32), i2f.astype(jnp.int32)], axis=0
    )


def kernel(input, W, b):
    n_tok = input.shape[0]
    grid = (n_tok // _BM,)
    b2d = b.reshape(_NUM_OUT, 1)
    probs_t, idx_t = pl.pallas_call(
        _router_block,
        grid=grid,
        in_specs=[
            pl.BlockSpec((_BM, _DIM), lambda i: (i, 0)),
            pl.BlockSpec((_NUM_OUT, _DIM), lambda i: (0, 0)),
            pl.BlockSpec((_NUM_OUT, 1), lambda i: (0, 0)),
        ],
        out_specs=[
            pl.BlockSpec((2, _BM), lambda i: (0, i)),
            pl.BlockSpec((2, _BM), lambda i: (0, i)),
        ],
        out_shape=[
            jax.ShapeDtypeStruct((2, n_tok), jnp.float32),
            jax.ShapeDtypeStruct((2, n_tok), jnp.int32),
        ],
        compiler_params=pltpu.CompilerParams(
            dimension_semantics=("parallel",),
        ),
    )(input, W, b2d)
    return probs_t.T, idx_t.T
